# baseline (speedup 1.0000x reference)
import functools

import jax
import jax.numpy as jnp
from jax.experimental import pallas as pl

N = 50000
E = 800000

_INTERPRET = False

TE = 2000


def _silu(x):
    return x * jax.nn.sigmoid(x)


def _edge_kernel(xr_ref, xc_ref, ce_ref,
                 A_ref, B_ref, wre_ref, be1_ref,
                 We2_ref, be2_ref, Wc1_ref, bc1_ref, Wc2_ref,
                 m_ref, tr_ref):
    xr = xr_ref[...]
    xc = xc_ref[...]
    ce = ce_ref[...]
    lane = jax.lax.broadcasted_iota(jnp.int32, ce.shape, 1)
    mask3 = lane < 3
    cd = jnp.where(mask3, ce, 0.0)
    radial = jnp.sum(cd * cd, axis=1, keepdims=True)
    ea = ce[:, 3:4]
    rad_ea = jnp.concatenate([radial, ea], axis=1)
    pre = (jnp.dot(xr, A_ref[...], preferred_element_type=jnp.float32)
           + jnp.dot(xc, B_ref[...], preferred_element_type=jnp.float32)
           + jnp.dot(rad_ea, wre_ref[...], preferred_element_type=jnp.float32)
           + be1_ref[...])
    m = _silu(pre)
    m = _silu(jnp.dot(m, We2_ref[...], preferred_element_type=jnp.float32)
              + be2_ref[...])
    tt = _silu(jnp.dot(m, Wc1_ref[...], preferred_element_type=jnp.float32)
               + bc1_ref[...])
    t = jnp.dot(tt, Wc2_ref[...], preferred_element_type=jnp.float32)
    m_ref[...] = m
    tr_ref[...] = jnp.where(mask3, ce * t, 1.0)


def _edge_mlp(xr, xc, ce, p):
    hid = p['We2'].shape[0]
    inf = (p['We1'].shape[0] - 2) // 2
    A = p['We1'][:inf]
    B = p['We1'][inf:2 * inf]
    wre = p['We1'][2 * inf:]
    grid = (E // TE,)
    erow = lambda i: (i, 0)
    wfull = lambda i: (0, 0)
    out = pl.pallas_call(
        _edge_kernel,
        grid=grid,
        in_specs=[
            pl.BlockSpec((TE, inf), erow),
            pl.BlockSpec((TE, inf), erow),
            pl.BlockSpec((TE, 4), erow),
            pl.BlockSpec(A.shape, wfull),
            pl.BlockSpec(B.shape, wfull),
            pl.BlockSpec(wre.shape, wfull),
            pl.BlockSpec((1, hid), wfull),
            pl.BlockSpec(p['We2'].shape, wfull),
            pl.BlockSpec((1, hid), wfull),
            pl.BlockSpec(p['Wc1'].shape, wfull),
            pl.BlockSpec((1, hid), wfull),
            pl.BlockSpec(p['Wc2'].shape, wfull),
        ],
        out_specs=[
            pl.BlockSpec((TE, hid), erow),
            pl.BlockSpec((TE, 4), erow),
        ],
        out_shape=[
            jax.ShapeDtypeStruct((E, hid), jnp.float32),
            jax.ShapeDtypeStruct((E, 4), jnp.float32),
        ],
        interpret=_INTERPRET,
    )(xr, xc, ce,
      A, B, wre, p['be1'][None, :],
      p['We2'], p['be2'][None, :], p['Wc1'], p['bc1'][None, :], p['Wc2'])
    return out


def _segment_sum(data, seg, num):
    return jax.ops.segment_sum(data, seg, num_segments=num)


def kernel(pos, edge_attr, params, edge_index, face, vertex2face, batch, ptr,
           face_len, vertex2face_len):
    row, col = edge_index[0], edge_index[1]

    centroid = jnp.mean(pos, axis=0, keepdims=True)
    p = pos - centroid
    mx = jnp.max(jnp.sqrt(jnp.sum(p ** 2, axis=1)))
    p = p / mx

    v0 = p[face[0]]
    v1 = p[face[1]]
    v2 = p[face[2]]
    fn = jnp.cross(v1 - v0, v2 - v0)
    face_area = jnp.sqrt(jnp.sum(fn ** 2, axis=1)) / 2.0
    vtx = vertex2face[:, 0]
    fidx = vertex2face[:, 1]
    asum = _segment_sum(face_area[fidx], vtx, N)
    acnt = jnp.maximum(_segment_sum(jnp.ones((vtx.shape[0],), jnp.float32), vtx, N), 1.0)
    area = asum / acnt
    x = area[:, None] * params['feat_W'][0][None, :] + params['feat_b'][None, :]

    coord = p
    for lp in (params['c1'], params['c2'], params['c3']):
        xr = x[row]
        xc = x[col]
        cd = coord[row] - coord[col]
        ce = jnp.concatenate([cd, edge_attr], axis=1)
        m, tr = _edge_mlp(xr, xc, ce, lp)
        agg = _segment_sum(jnp.concatenate([m, tr], axis=1), row, N)
        hid = m.shape[1]
        magg = agg[:, :hid]
        cnt = jnp.maximum(agg[:, hid + 3], 1.0)
        coord = coord + agg[:, hid:hid + 3] / cnt[:, None]
        h = jnp.concatenate([x, magg], axis=1)
        h = _silu(h @ lp['Wn1'] + lp['bn1'])
        x = h @ lp['Wn2'] + lp['bn2']

    x = jax.nn.relu(x @ params['lin1_W'] + params['lin1_b'])
    x = jnp.mean(x, axis=0, keepdims=True)
    x = x @ params['lin2_W'] + params['lin2_b']
    return jax.nn.log_softmax(x, axis=1)


# speedup vs baseline: 2.3341x; 2.3341x over previous
import functools

import jax
import jax.numpy as jnp
from jax import lax
from jax.experimental import pallas as pl
from jax.experimental.pallas import tpu as pltpu
from jax.experimental.pallas import tpu_sc as plsc

N = 50000
E = 800000

_INTERPRET = False

TE = 2000
NC, NS = 2, 16
NW = NC * NS
PER_W = E // NW
GC = 1000


def _silu(x):
    return x * jax.nn.sigmoid(x)


def _gather_add(tab_r, tab_c, row, col):
    P = tab_r.shape[1]
    mesh = plsc.VectorSubcoreMesh(core_axis_name="c", subcore_axis_name="s")

    @functools.partial(
        pl.kernel,
        out_type=jax.ShapeDtypeStruct((E, P), jnp.float32),
        mesh=mesh,
        scratch_types=[
            pltpu.VMEM((GC,), jnp.int32),
            pltpu.VMEM((GC,), jnp.int32),
            pltpu.VMEM((GC, P), jnp.float32),
            pltpu.SemaphoreType.DMA,
        ],
    )
    def k(tab_r_hbm, tab_c_hbm, row_hbm, col_hbm, out_hbm, ridx, cidx, buf, sem):
        wid = lax.axis_index("s") * NC + lax.axis_index("c")
        base = wid * PER_W

        def body(i, carry):
            off = base + i * GC
            pltpu.sync_copy(row_hbm.at[pl.ds(off, GC)], ridx)
            pltpu.sync_copy(col_hbm.at[pl.ds(off, GC)], cidx)
            pltpu.async_copy(tab_r_hbm.at[ridx], buf, sem).wait()
            pltpu.async_copy(tab_c_hbm.at[cidx], buf, sem, add=True).wait()
            pltpu.sync_copy(buf, out_hbm.at[pl.ds(off, GC)])
            return carry

        lax.fori_loop(0, PER_W // GC, body, 0)

    return k(tab_r, tab_c, row, col)


def _edge_kernel(hid, g_ref, ea_ref, wre_ref,
                 We2_ref, be2_ref, Wc1_ref, bc1_ref, Wc2_ref,
                 m_ref, tr_ref):
    g = g_ref[...]
    pre = g[:, :hid]
    cd = g[:, hid:hid + 3]
    radial = jnp.sum(cd * cd, axis=1, keepdims=True)
    ea = ea_ref[...]
    rad_ea = jnp.concatenate([radial, ea], axis=1)
    pre = pre + jnp.dot(rad_ea, wre_ref[...], preferred_element_type=jnp.float32)
    m = _silu(pre)
    m = _silu(jnp.dot(m, We2_ref[...], preferred_element_type=jnp.float32)
              + be2_ref[...])
    tt = _silu(jnp.dot(m, Wc1_ref[...], preferred_element_type=jnp.float32)
               + bc1_ref[...])
    t = jnp.dot(tt, Wc2_ref[...], preferred_element_type=jnp.float32)
    m_ref[...] = m
    ones = jnp.ones_like(t)
    tr_ref[...] = jnp.concatenate([cd * t, ones], axis=1)


def _edge_mlp(g, edge_attr, p):
    hid = p['We2'].shape[0]
    inf = (p['We1'].shape[0] - 2) // 2
    wre = p['We1'][2 * inf:]
    P = g.shape[1]
    grid = (E // TE,)
    erow = lambda i: (i, 0)
    wfull = lambda i: (0, 0)
    out = pl.pallas_call(
        functools.partial(_edge_kernel, hid),
        grid=grid,
        in_specs=[
            pl.BlockSpec((TE, P), erow),
            pl.BlockSpec((TE, 1), erow),
            pl.BlockSpec(wre.shape, wfull),
            pl.BlockSpec(p['We2'].shape, wfull),
            pl.BlockSpec((1, hid), wfull),
            pl.BlockSpec(p['Wc1'].shape, wfull),
            pl.BlockSpec((1, hid), wfull),
            pl.BlockSpec(p['Wc2'].shape, wfull),
        ],
        out_specs=[
            pl.BlockSpec((TE, hid), erow),
            pl.BlockSpec((TE, 4), erow),
        ],
        out_shape=[
            jax.ShapeDtypeStruct((E, hid), jnp.float32),
            jax.ShapeDtypeStruct((E, 4), jnp.float32),
        ],
        interpret=_INTERPRET,
    )(g, edge_attr,
      wre, p['We2'], p['be2'][None, :], p['Wc1'], p['bc1'][None, :], p['Wc2'])
    return out


def _segment_sum(data, seg, num):
    return jax.ops.segment_sum(data, seg, num_segments=num)


def kernel(pos, edge_attr, params, edge_index, face, vertex2face, batch, ptr,
           face_len, vertex2face_len):
    row, col = edge_index[0], edge_index[1]

    centroid = jnp.mean(pos, axis=0, keepdims=True)
    p = pos - centroid
    mx = jnp.max(jnp.sqrt(jnp.sum(p ** 2, axis=1)))
    p = p / mx

    v0 = p[face[0]]
    v1 = p[face[1]]
    v2 = p[face[2]]
    fn = jnp.cross(v1 - v0, v2 - v0)
    face_area = jnp.sqrt(jnp.sum(fn ** 2, axis=1)) / 2.0
    vtx = vertex2face[:, 0]
    fidx = vertex2face[:, 1]
    asum = _segment_sum(face_area[fidx], vtx, N)
    acnt = jnp.maximum(_segment_sum(jnp.ones((vtx.shape[0],), jnp.float32), vtx, N), 1.0)
    area = asum / acnt
    x = area[:, None] * params['feat_W'][0][None, :] + params['feat_b'][None, :]

    coord = p
    for lp in (params['c1'], params['c2'], params['c3']):
        hid = lp['We2'].shape[0]
        inf = (lp['We1'].shape[0] - 2) // 2
        A = lp['We1'][:inf]
        B = lp['We1'][inf:2 * inf]
        P = 128
        pad = jnp.zeros((N, P - hid - 3), jnp.float32)
        tab_r = jnp.concatenate([x @ A + lp['be1'][None, :], coord, pad], axis=1)
        tab_c = jnp.concatenate([x @ B, -coord, pad], axis=1)
        g = _gather_add(tab_r, tab_c, row, col)
        m, tr = _edge_mlp(g, edge_attr, lp)
        agg = _segment_sum(jnp.concatenate([m, tr], axis=1), row, N)
        magg = agg[:, :hid]
        cnt = jnp.maximum(agg[:, hid + 3], 1.0)
        coord = coord + agg[:, hid:hid + 3] / cnt[:, None]
        h = jnp.concatenate([x, magg], axis=1)
        h = _silu(h @ lp['Wn1'] + lp['bn1'])
        x = h @ lp['Wn2'] + lp['bn2']

    x = jax.nn.relu(x @ params['lin1_W'] + params['lin1_b'])
    x = jnp.mean(x, axis=0, keepdims=True)
    x = x @ params['lin2_W'] + params['lin2_b']
    return jax.nn.log_softmax(x, axis=1)


# speedup vs baseline: 2.6047x; 1.1159x over previous
import functools

import jax
import jax.numpy as jnp
from jax import lax
from jax.experimental import pallas as pl
from jax.experimental.pallas import tpu as pltpu
from jax.experimental.pallas import tpu_sc as plsc

N = 50000
E = 800000

_INTERPRET = False

TE = 2000
NC, NS = 2, 16
NW = NC * NS
PER_W = E // NW
GC = 1000


def _silu(x):
    return x * jax.nn.sigmoid(x)


def _gather_add(tab_r, tab_c, row, col):
    P = tab_r.shape[1]
    mesh = plsc.VectorSubcoreMesh(core_axis_name="c", subcore_axis_name="s")

    @functools.partial(
        pl.kernel,
        out_type=jax.ShapeDtypeStruct((E, P), jnp.float32),
        mesh=mesh,
        scratch_types=[
            pltpu.VMEM((GC,), jnp.int32),
            pltpu.VMEM((GC,), jnp.int32),
            pltpu.VMEM((GC, P), jnp.float32),
            pltpu.SemaphoreType.DMA,
        ],
    )
    def k(tab_r_hbm, tab_c_hbm, row_hbm, col_hbm, out_hbm, ridx, cidx, buf, sem):
        wid = lax.axis_index("s") * NC + lax.axis_index("c")
        base = wid * PER_W

        def body(i, carry):
            off = base + i * GC
            pltpu.sync_copy(row_hbm.at[pl.ds(off, GC)], ridx)
            pltpu.sync_copy(col_hbm.at[pl.ds(off, GC)], cidx)
            pltpu.async_copy(tab_r_hbm.at[ridx], buf, sem).wait()
            pltpu.async_copy(tab_c_hbm.at[cidx], buf, sem, add=True).wait()
            pltpu.sync_copy(buf, out_hbm.at[pl.ds(off, GC)])
            return carry

        lax.fori_loop(0, PER_W // GC, body, 0)

    return k(tab_r, tab_c, row, col)


SC_C = 1000
STRIPE = 3200
LAST = N - 15 * STRIPE


def _scatter_add(mt, row, Wh):
    mesh = plsc.VectorSubcoreMesh(core_axis_name="c", subcore_axis_name="s")
    EPT = E // NS

    @functools.partial(
        pl.kernel,
        out_type=jax.ShapeDtypeStruct((2, N, Wh), jnp.float32),
        mesh=mesh,
        scratch_types=[
            pltpu.VMEM((SC_C,), jnp.int32),
            pltpu.VMEM((SC_C, 2 * Wh), jnp.float32),
            pltpu.VMEM_SHARED((N, Wh), jnp.float32),
            pltpu.SemaphoreType.DMA,
        ],
    )
    def k(mt_hbm, row_hbm, z_hbm, out_hbm, ridx, mbuf, acc, sem):
        cc = lax.axis_index("c")
        s = lax.axis_index("s")

        @pl.when(s < 15)
        def _():
            pltpu.sync_copy(z_hbm.at[pl.ds(s * STRIPE, STRIPE)],
                            acc.at[pl.ds(s * STRIPE, STRIPE)])

        @pl.when(s == 15)
        def _():
            pltpu.sync_copy(z_hbm.at[pl.ds(15 * STRIPE, LAST)],
                            acc.at[pl.ds(15 * STRIPE, LAST)])

        plsc.subcore_barrier()
        base = s * EPT

        def body(i, carry):
            off = base + i * SC_C
            pltpu.sync_copy(row_hbm.at[pl.ds(off, SC_C)], ridx)
            pltpu.sync_copy(mt_hbm.at[pl.ds(off, SC_C)], mbuf)

            @pl.when(cc == 0)
            def _():
                pltpu.async_copy(mbuf.at[:, pl.ds(0, Wh)], acc.at[ridx],
                                 sem, add=True).wait()

            @pl.when(cc == 1)
            def _():
                pltpu.async_copy(mbuf.at[:, pl.ds(Wh, Wh)], acc.at[ridx],
                                 sem, add=True).wait()

            return carry

        lax.fori_loop(0, EPT // SC_C, body, 0)
        plsc.subcore_barrier()

        @pl.when(s < 15)
        def _():
            pltpu.sync_copy(acc.at[pl.ds(s * STRIPE, STRIPE)],
                            out_hbm.at[cc, pl.ds(s * STRIPE, STRIPE)])

        @pl.when(s == 15)
        def _():
            pltpu.sync_copy(acc.at[pl.ds(15 * STRIPE, LAST)],
                            out_hbm.at[cc, pl.ds(15 * STRIPE, LAST)])

    return k(mt, row, jnp.zeros((N, Wh), jnp.float32))


def _edge_kernel(hid, Wh, g_ref, ea_ref, wre_ref,
                 We2_ref, be2_ref, Wc1_ref, bc1_ref, Wc2_ref,
                 mt_ref):
    g = g_ref[...]
    pre = g[:, :hid]
    cd = g[:, hid:hid + 3]
    radial = jnp.sum(cd * cd, axis=1, keepdims=True)
    ea = ea_ref[...]
    rad_ea = jnp.concatenate([radial, ea], axis=1)
    pre = pre + jnp.dot(rad_ea, wre_ref[...], preferred_element_type=jnp.float32)
    m = _silu(pre)
    m = _silu(jnp.dot(m, We2_ref[...], preferred_element_type=jnp.float32)
              + be2_ref[...])
    tt = _silu(jnp.dot(m, Wc1_ref[...], preferred_element_type=jnp.float32)
               + bc1_ref[...])
    t = jnp.dot(tt, Wc2_ref[...], preferred_element_type=jnp.float32)
    ones = jnp.ones_like(t)
    pad = jnp.zeros((m.shape[0], 2 * Wh - hid - 4), jnp.float32)
    mt_ref[...] = jnp.concatenate([m, cd * t, ones, pad], axis=1)


def _edge_mlp(g, edge_attr, p, Wh):
    hid = p['We2'].shape[0]
    inf = (p['We1'].shape[0] - 2) // 2
    wre = p['We1'][2 * inf:]
    P = g.shape[1]
    grid = (E // TE,)
    erow = lambda i: (i, 0)
    wfull = lambda i: (0, 0)
    out = pl.pallas_call(
        functools.partial(_edge_kernel, hid, Wh),
        grid=grid,
        in_specs=[
            pl.BlockSpec((TE, P), erow),
            pl.BlockSpec((TE, 1), erow),
            pl.BlockSpec(wre.shape, wfull),
            pl.BlockSpec(p['We2'].shape, wfull),
            pl.BlockSpec((1, hid), wfull),
            pl.BlockSpec(p['Wc1'].shape, wfull),
            pl.BlockSpec((1, hid), wfull),
            pl.BlockSpec(p['Wc2'].shape, wfull),
        ],
        out_specs=[
            pl.BlockSpec((TE, 2 * Wh), erow),
        ],
        out_shape=[
            jax.ShapeDtypeStruct((E, 2 * Wh), jnp.float32),
        ],
        interpret=_INTERPRET,
    )(g, edge_attr,
      wre, p['We2'], p['be2'][None, :], p['Wc1'], p['bc1'][None, :], p['Wc2'])
    return out[0]


def _segment_sum(data, seg, num):
    return jax.ops.segment_sum(data, seg, num_segments=num)


def kernel(pos, edge_attr, params, edge_index, face, vertex2face, batch, ptr,
           face_len, vertex2face_len):
    row, col = edge_index[0], edge_index[1]

    centroid = jnp.mean(pos, axis=0, keepdims=True)
    p = pos - centroid
    mx = jnp.max(jnp.sqrt(jnp.sum(p ** 2, axis=1)))
    p = p / mx

    v0 = p[face[0]]
    v1 = p[face[1]]
    v2 = p[face[2]]
    fn = jnp.cross(v1 - v0, v2 - v0)
    face_area = jnp.sqrt(jnp.sum(fn ** 2, axis=1)) / 2.0
    vtx = vertex2face[:, 0]
    fidx = vertex2face[:, 1]
    asum = _segment_sum(face_area[fidx], vtx, N)
    acnt = jnp.maximum(_segment_sum(jnp.ones((vtx.shape[0],), jnp.float32), vtx, N), 1.0)
    area = asum / acnt
    x = area[:, None] * params['feat_W'][0][None, :] + params['feat_b'][None, :]

    coord = p
    for lp in (params['c1'], params['c2'], params['c3']):
        hid = lp['We2'].shape[0]
        inf = (lp['We1'].shape[0] - 2) // 2
        A = lp['We1'][:inf]
        B = lp['We1'][inf:2 * inf]
        P = 128
        pad = jnp.zeros((N, P - hid - 3), jnp.float32)
        tab_r = jnp.concatenate([x @ A + lp['be1'][None, :], coord, pad], axis=1)
        tab_c = jnp.concatenate([x @ B, -coord, pad], axis=1)
        g = _gather_add(tab_r, tab_c, row, col)
        Wh = {16: 12, 32: 20, 64: 36}[hid]
        mt = _edge_mlp(g, edge_attr, lp, Wh)
        agg = _segment_sum(mt, row, N)
        magg = agg[:, :hid]
        cnt = jnp.maximum(agg[:, hid + 3], 1.0)
        coord = coord + agg[:, hid:hid + 3] / cnt[:, None]
        h = jnp.concatenate([x, magg], axis=1)
        h = _silu(h @ lp['Wn1'] + lp['bn1'])
        x = h @ lp['Wn2'] + lp['bn2']

    x = jax.nn.relu(x @ params['lin1_W'] + params['lin1_b'])
    x = jnp.mean(x, axis=0, keepdims=True)
    x = x @ params['lin2_W'] + params['lin2_b']
    return jax.nn.log_softmax(x, axis=1)
